# baseline (speedup 1.0000x reference)
import functools

import jax
import jax.numpy as jnp
from jax import lax
from jax.experimental import pallas as pl
from jax.experimental.pallas import tpu as pltpu
from jax.experimental.pallas import tpu_sc as plsc


def _select_block_kernel(pts_ref, vT_ref, idx_ref, *, K):
    pts_b = pts_ref[...]
    vT = vT_ref[...]
    BN = pts_b.shape[0]
    V = vT.shape[1]

    vnorm2 = jnp.sum(vT * vT, axis=0, keepdims=True)
    pnorm2 = jnp.sum(pts_b * pts_b, axis=1, keepdims=True)

    dots_sel = jnp.dot(pts_b, vT, preferred_element_type=jnp.float32)
    work = (pnorm2 - 2.0 * dots_sel) + vnorm2

    iota = lax.broadcasted_iota(jnp.int32, (BN, V), 1)
    BIG = jnp.float32(3.0e38)
    amins = []
    for _ in range(K):
        amin = jnp.argmin(work, axis=1).astype(jnp.int32)[:, None]
        work = jnp.where(iota == amin, BIG, work)
        amins.append(amin)
    idx_ref[...] = jnp.concatenate(amins, axis=1)


def _rsqrt_nr(x):
    halfx = 0.5 * x
    i = jax.lax.bitcast_convert_type(x, jnp.int32)
    i = jnp.int32(0x5F3759DF) - (i >> 1)
    y = jax.lax.bitcast_convert_type(i, jnp.float32)
    for _ in range(4):
        y = y * (1.5 - (halfx * y) * y)
    return y


def _make_sc_blend(N, K, NC, NS):
    NW = NC * NS
    pts_per_w = N // NW
    groups = pts_per_w // 16
    mesh = plsc.VectorSubcoreMesh(core_axis_name="c", subcore_axis_name="s")
    fdt = jnp.float32

    @functools.partial(
        pl.kernel,
        mesh=mesh,
        compiler_params=pltpu.CompilerParams(needs_layout_passes=False),
        out_type=[jax.ShapeDtypeStruct((N,), fdt) for _ in range(4)],
        scratch_types=[
            pltpu.VMEM((16384,), fdt),
            pltpu.VMEM((16384,), fdt),
            pltpu.VMEM((16384,), fdt),
            pltpu.VMEM((16384,), fdt),
            pltpu.VMEM((16384,), fdt),
            pltpu.VMEM((16384,), fdt),
            pltpu.VMEM((pts_per_w * K,), jnp.int32),
            pltpu.VMEM((pts_per_w,), fdt),
            pltpu.VMEM((pts_per_w,), fdt),
            pltpu.VMEM((pts_per_w,), fdt),
            pltpu.VMEM((16,), fdt),
            pltpu.VMEM((pts_per_w,), fdt),
            pltpu.VMEM((pts_per_w,), fdt),
            pltpu.VMEM((pts_per_w,), fdt),
            pltpu.VMEM((pts_per_w,), fdt),
        ],
    )
    def sc_blend(
        vx_h, vy_h, vz_h, nx_h, ny_h, nz_h, idx_h, px_h, py_h, pz_h, s_h,
        sdf_h, onx_h, ony_h, onz_h,
        vx_v, vy_v, vz_v, nx_v, ny_v, nz_v, idx_v, px_v, py_v, pz_v, s_v,
        osdf_v, ox_v, oy_v, oz_v,
    ):
        wid = lax.axis_index("s") * NC + lax.axis_index("c")
        base = wid * pts_per_w

        pltpu.sync_copy(vx_h, vx_v)
        pltpu.sync_copy(vy_h, vy_v)
        pltpu.sync_copy(vz_h, vz_v)
        pltpu.sync_copy(nx_h, nx_v)
        pltpu.sync_copy(ny_h, ny_v)
        pltpu.sync_copy(nz_h, nz_v)
        pltpu.sync_copy(idx_h.at[pl.ds(base * K, pts_per_w * K)], idx_v)
        pltpu.sync_copy(px_h.at[pl.ds(base, pts_per_w)], px_v)
        pltpu.sync_copy(py_h.at[pl.ds(base, pts_per_w)], py_v)
        pltpu.sync_copy(pz_h.at[pl.ds(base, pts_per_w)], pz_v)
        pltpu.sync_copy(s_h, s_v)

        s = s_v[...]
        lane = lax.broadcasted_iota(jnp.int32, (16,), 0)

        for g in range(groups):
            px = px_v[pl.ds(g * 16, 16)]
            py = py_v[pl.ds(g * 16, 16)]
            pz = pz_v[pl.ds(g * 16, 16)]
            num = jnp.zeros((16,), fdt)
            den = jnp.zeros((16,), fdt)
            anx = jnp.zeros((16,), fdt)
            any_ = jnp.zeros((16,), fdt)
            anz = jnp.zeros((16,), fdt)
            for k in range(K):
                lanes = lane * K + (g * 16 * K + k)
                iv = plsc.load_gather(idx_v, [lanes])
                gvx = plsc.load_gather(vx_v, [iv])
                gvy = plsc.load_gather(vy_v, [iv])
                gvz = plsc.load_gather(vz_v, [iv])
                gnx = plsc.load_gather(nx_v, [iv])
                gny = plsc.load_gather(ny_v, [iv])
                gnz = plsc.load_gather(nz_v, [iv])

                nn2 = gnx * gnx + gny * gny + gnz * gnz
                rs = _rsqrt_nr(nn2)
                gnx, gny, gnz = gnx * rs, gny * rs, gnz * rs

                ex, ey, ez = px - gvx, py - gvy, pz - gvz
                d2 = ex * ex + ey * ey + ez * ez
                rsd = _rsqrt_nr(d2)
                dist = d2 * rsd
                dot = ex * gnx + ey * gny + ez * gnz
                w_d = 1.0 / (dist + 1e-5)
                w_p = jnp.minimum(dist, jnp.exp(-s * w_d))
                pf = jnp.where(dot < 0, -dist, dist)
                h = (0.1 * dot + w_p * pf) / (w_p + (0.1 + 1e-5))
                num = num + w_d * h
                den = den + w_d
                anx = anx + w_d * gnx
                any_ = any_ + w_d * gny
                anz = anz + w_d * gnz

            osdf_v[pl.ds(g * 16, 16)] = num / den
            onorm2 = anx * anx + any_ * any_ + anz * anz
            rso = _rsqrt_nr(onorm2)
            ox_v[pl.ds(g * 16, 16)] = anx * rso
            oy_v[pl.ds(g * 16, 16)] = any_ * rso
            oz_v[pl.ds(g * 16, 16)] = anz * rso

        pltpu.sync_copy(osdf_v, sdf_h.at[pl.ds(base, pts_per_w)])
        pltpu.sync_copy(ox_v, onx_h.at[pl.ds(base, pts_per_w)])
        pltpu.sync_copy(oy_v, ony_h.at[pl.ds(base, pts_per_w)])
        pltpu.sync_copy(oz_v, onz_h.at[pl.ds(base, pts_per_w)])

    return sc_blend


def kernel(pts, vertices, vert_normals, s):
    if pts.ndim < 3:
        pts = pts[None]
    B, N, _ = pts.shape
    V = vertices.shape[0]
    K = 8

    pts2d = pts.reshape(B * N, 3).astype(jnp.float32)
    vT = vertices.T.astype(jnp.float32)

    BN = 256
    while (B * N) % BN:
        BN //= 2
    grid = ((B * N) // BN,)

    idx = pl.pallas_call(
        functools.partial(_select_block_kernel, K=K),
        grid=grid,
        in_specs=[
            pl.BlockSpec((BN, 3), lambda i: (i, 0)),
            pl.BlockSpec((3, V), lambda i: (0, 0)),
        ],
        out_specs=pl.BlockSpec((BN, K), lambda i: (i, 0)),
        out_shape=jax.ShapeDtypeStruct((B * N, K), jnp.int32),
    )(pts2d, vT)

    info = plsc.get_sparse_core_info()
    NC, NS = info.num_cores, info.num_subcores
    sc_blend = _make_sc_blend(B * N, K, NC, NS)
    vf = vertices.astype(jnp.float32)
    nf = vert_normals.astype(jnp.float32)
    s_arr = jnp.full((16,), jnp.asarray(s, jnp.float32))
    sdf1d, onx, ony, onz = sc_blend(
        vf[:, 0], vf[:, 1], vf[:, 2],
        nf[:, 0], nf[:, 1], nf[:, 2],
        idx.reshape(-1),
        pts2d[:, 0], pts2d[:, 1], pts2d[:, 2],
        s_arr,
    )

    sdf = sdf1d.reshape(B, N)
    normals = jnp.stack([onx, ony, onz], axis=-1).reshape(B, N, 3)
    return sdf, normals


# speedup vs baseline: 1.0057x; 1.0057x over previous
import functools

import jax
import jax.numpy as jnp
from jax import lax
from jax.experimental import pallas as pl
from jax.experimental.pallas import tpu as pltpu
from jax.experimental.pallas import tpu_sc as plsc


def _select_block_kernel(pts_ref, vT_ref, idx_ref, *, K):
    pts_b = pts_ref[...]
    vT = vT_ref[...]
    BN = pts_b.shape[0]
    V = vT.shape[1]

    vnorm2 = jnp.sum(vT * vT, axis=0, keepdims=True)
    pnorm2 = jnp.sum(pts_b * pts_b, axis=1, keepdims=True)

    dots_sel = jnp.dot(pts_b, vT, preferred_element_type=jnp.float32)
    work = (pnorm2 - 2.0 * dots_sel) + vnorm2

    iota = lax.broadcasted_iota(jnp.int32, (BN, V), 1)
    BIG = jnp.float32(3.0e38)
    amins = []
    for _ in range(K):
        amin = jnp.argmin(work, axis=1).astype(jnp.int32)[:, None]
        work = jnp.where(iota == amin, BIG, work)
        amins.append(amin)
    idx_ref[...] = jnp.concatenate(amins, axis=1)


def _rsqrt_nr(x):
    halfx = 0.5 * x
    i = jax.lax.bitcast_convert_type(x, jnp.int32)
    i = jnp.int32(0x5F3759DF) - (i >> 1)
    y = jax.lax.bitcast_convert_type(i, jnp.float32)
    for _ in range(4):
        y = y * (1.5 - (halfx * y) * y)
    return y


def _make_sc_blend(N, K, NC, NS):
    NW = NC * NS
    pts_per_w = N // NW
    groups = pts_per_w // 16
    mesh = plsc.VectorSubcoreMesh(core_axis_name="c", subcore_axis_name="s")
    fdt = jnp.float32

    @functools.partial(
        pl.kernel,
        mesh=mesh,
        compiler_params=pltpu.CompilerParams(needs_layout_passes=False),
        out_type=[jax.ShapeDtypeStruct((N,), fdt) for _ in range(4)],
        scratch_types=[
            pltpu.VMEM((16384,), fdt),
            pltpu.VMEM((16384,), fdt),
            pltpu.VMEM((16384,), fdt),
            pltpu.VMEM((16384,), fdt),
            pltpu.VMEM((16384,), fdt),
            pltpu.VMEM((16384,), fdt),
            pltpu.VMEM((pts_per_w * K,), jnp.int32),
            pltpu.VMEM((pts_per_w,), fdt),
            pltpu.VMEM((pts_per_w,), fdt),
            pltpu.VMEM((pts_per_w,), fdt),
            pltpu.VMEM((16,), fdt),
            pltpu.VMEM((pts_per_w,), fdt),
            pltpu.VMEM((pts_per_w,), fdt),
            pltpu.VMEM((pts_per_w,), fdt),
            pltpu.VMEM((pts_per_w,), fdt),
        ],
    )
    def sc_blend(
        vx_h, vy_h, vz_h, nx_h, ny_h, nz_h, idx_h, px_h, py_h, pz_h, s_h,
        sdf_h, onx_h, ony_h, onz_h,
        vx_v, vy_v, vz_v, nx_v, ny_v, nz_v, idx_v, px_v, py_v, pz_v, s_v,
        osdf_v, ox_v, oy_v, oz_v,
    ):
        wid = lax.axis_index("s") * NC + lax.axis_index("c")
        base = wid * pts_per_w

        pltpu.sync_copy(vx_h, vx_v)
        pltpu.sync_copy(vy_h, vy_v)
        pltpu.sync_copy(vz_h, vz_v)
        pltpu.sync_copy(nx_h, nx_v)
        pltpu.sync_copy(ny_h, ny_v)
        pltpu.sync_copy(nz_h, nz_v)
        pltpu.sync_copy(idx_h.at[pl.ds(base * K, pts_per_w * K)], idx_v)
        pltpu.sync_copy(px_h.at[pl.ds(base, pts_per_w)], px_v)
        pltpu.sync_copy(py_h.at[pl.ds(base, pts_per_w)], py_v)
        pltpu.sync_copy(pz_h.at[pl.ds(base, pts_per_w)], pz_v)
        pltpu.sync_copy(s_h, s_v)

        s = s_v[...]
        lane = lax.broadcasted_iota(jnp.int32, (16,), 0)

        for g in range(groups):
            px = px_v[pl.ds(g * 16, 16)]
            py = py_v[pl.ds(g * 16, 16)]
            pz = pz_v[pl.ds(g * 16, 16)]
            num = jnp.zeros((16,), fdt)
            den = jnp.zeros((16,), fdt)
            anx = jnp.zeros((16,), fdt)
            any_ = jnp.zeros((16,), fdt)
            anz = jnp.zeros((16,), fdt)
            for k in range(K):
                lanes = lane * K + (g * 16 * K + k)
                iv = plsc.load_gather(idx_v, [lanes])
                gvx = plsc.load_gather(vx_v, [iv])
                gvy = plsc.load_gather(vy_v, [iv])
                gvz = plsc.load_gather(vz_v, [iv])
                gnx = plsc.load_gather(nx_v, [iv])
                gny = plsc.load_gather(ny_v, [iv])
                gnz = plsc.load_gather(nz_v, [iv])

                nn2 = gnx * gnx + gny * gny + gnz * gnz
                rs = _rsqrt_nr(nn2)
                gnx, gny, gnz = gnx * rs, gny * rs, gnz * rs

                ex, ey, ez = px - gvx, py - gvy, pz - gvz
                d2 = ex * ex + ey * ey + ez * ez
                rsd = _rsqrt_nr(d2)
                dist = d2 * rsd
                dot = ex * gnx + ey * gny + ez * gnz
                w_d = 1.0 / (dist + 1e-5)
                w_p = jnp.minimum(dist, jnp.exp(-s * w_d))
                pf = jnp.where(dot < 0, -dist, dist)
                h = (0.1 * dot + w_p * pf) / (w_p + (0.1 + 1e-5))
                num = num + w_d * h
                den = den + w_d
                anx = anx + w_d * gnx
                any_ = any_ + w_d * gny
                anz = anz + w_d * gnz

            osdf_v[pl.ds(g * 16, 16)] = num / den
            onorm2 = anx * anx + any_ * any_ + anz * anz
            rso = _rsqrt_nr(onorm2)
            ox_v[pl.ds(g * 16, 16)] = anx * rso
            oy_v[pl.ds(g * 16, 16)] = any_ * rso
            oz_v[pl.ds(g * 16, 16)] = anz * rso

        pltpu.sync_copy(osdf_v, sdf_h.at[pl.ds(base, pts_per_w)])
        pltpu.sync_copy(ox_v, onx_h.at[pl.ds(base, pts_per_w)])
        pltpu.sync_copy(oy_v, ony_h.at[pl.ds(base, pts_per_w)])
        pltpu.sync_copy(oz_v, onz_h.at[pl.ds(base, pts_per_w)])

    return sc_blend


def kernel(pts, vertices, vert_normals, s):
    if pts.ndim < 3:
        pts = pts[None]
    B, N, _ = pts.shape
    V = vertices.shape[0]
    K = 8

    pts2d = pts.reshape(B * N, 3).astype(jnp.float32)
    vT = vertices.T.astype(jnp.float32)

    BN = 128
    while (B * N) % BN:
        BN //= 2
    grid = ((B * N) // BN,)

    idx = pl.pallas_call(
        functools.partial(_select_block_kernel, K=K),
        grid=grid,
        in_specs=[
            pl.BlockSpec((BN, 3), lambda i: (i, 0)),
            pl.BlockSpec((3, V), lambda i: (0, 0)),
        ],
        out_specs=pl.BlockSpec((BN, K), lambda i: (i, 0)),
        out_shape=jax.ShapeDtypeStruct((B * N, K), jnp.int32),
    )(pts2d, vT)

    info = plsc.get_sparse_core_info()
    NC, NS = info.num_cores, info.num_subcores
    sc_blend = _make_sc_blend(B * N, K, NC, NS)
    vf = vertices.astype(jnp.float32)
    nf = vert_normals.astype(jnp.float32)
    s_arr = jnp.full((16,), jnp.asarray(s, jnp.float32))
    sdf1d, onx, ony, onz = sc_blend(
        vf[:, 0], vf[:, 1], vf[:, 2],
        nf[:, 0], nf[:, 1], nf[:, 2],
        idx.reshape(-1),
        pts2d[:, 0], pts2d[:, 1], pts2d[:, 2],
        s_arr,
    )

    sdf = sdf1d.reshape(B, N)
    normals = jnp.stack([onx, ony, onz], axis=-1).reshape(B, N, 3)
    return sdf, normals
